# baseline (speedup 1.0000x reference)
import functools

import jax
import jax.numpy as jnp
from jax import lax
from jax.experimental import pallas as pl
from jax.experimental.pallas import tpu as pltpu
from jax.experimental.pallas import tpu_sc as plsc

N = 10000
E = 320000
D = 128

NC = 2
NS = 16
NW = NC * NS
B = 128
NB = 2
CT = NB * (-(-E // (NW * B * NB)))
EP = NW * B * CT
NP = 10240
RPT = NP // NS
L = 16

_MESH = plsc.VectorSubcoreMesh(core_axis_name="c", subcore_axis_name="s")


@functools.partial(
    pl.kernel,
    out_type=jax.ShapeDtypeStruct((NC, NP), jnp.float32),
    mesh=_MESH,
    scratch_types=[
        pltpu.VMEM((CT, B), jnp.int32),
        pltpu.VMEM((B,), jnp.float32),
        pltpu.VMEM((RPT,), jnp.float32),
        pltpu.VMEM_SHARED((NP,), jnp.float32),
    ],
)
def _sc_degree(dst_hbm, out_hbm, idx_v, ones_v, zrow_v, acc):
    c = lax.axis_index("c")
    s = lax.axis_index("s")
    wid = c * NS + s
    for i in range(B // L):
        ones_v[pl.ds(i * L, L)] = jnp.ones((L,), jnp.float32)
    for i in range(RPT // L):
        zrow_v[pl.ds(i * L, L)] = jnp.zeros((L,), jnp.float32)
    pltpu.sync_copy(zrow_v, acc.at[pl.ds(s * RPT, RPT)])
    pltpu.sync_copy(dst_hbm.at[wid], idx_v)
    plsc.subcore_barrier()

    def body(j, carry):
        pltpu.sync_copy(ones_v, acc.at[idx_v.at[j]], add=True)
        return carry

    lax.fori_loop(0, CT, body, 0)
    plsc.subcore_barrier()
    pltpu.sync_copy(acc.at[pl.ds(s * RPT, RPT)],
                    out_hbm.at[c, pl.ds(s * RPT, RPT)])


@functools.partial(
    pl.kernel,
    out_type=jax.ShapeDtypeStruct((NC, NP, D), jnp.float32),
    mesh=_MESH,
    scratch_types=[
        pltpu.VMEM((CT, B), jnp.int32),
        pltpu.VMEM((NB, B), jnp.int32),
        pltpu.VMEM((NB, B), jnp.int32),
        pltpu.VMEM((NB, B, D), jnp.float32),
        pltpu.VMEM_SHARED((NP, D), jnp.float32),
        pltpu.SemaphoreType.DMA,
        pltpu.SemaphoreType.DMA,
        pltpu.SemaphoreType.DMA,
        pltpu.SemaphoreType.DMA,
    ],
)
def _sc_prop(hs_hbm, pk_hbm, zeros_hbm, out_hbm,
             pk_v, src_v, dst_v, rows, acc,
             g0, g1, t0, t1):
    c = lax.axis_index("c")
    s = lax.axis_index("s")
    wid = c * NS + s
    gsem = (g0, g1)
    tsem = (t0, t1)

    pltpu.sync_copy(zeros_hbm, rows.at[0])
    for r in range(RPT // B):
        pltpu.sync_copy(rows.at[0], acc.at[pl.ds(s * RPT + r * B, B)])
    pltpu.sync_copy(pk_hbm.at[wid], pk_v)
    plsc.subcore_barrier()

    def unpack(j, slot):
        mask = jnp.full((L,), 0x3FFF, jnp.int32)
        sh = jnp.full((L,), 14, jnp.int32)
        for i in range(B // L):
            v = pk_v[j, pl.ds(i * L, L)]
            src_v[slot, pl.ds(i * L, L)] = lax.bitwise_and(v, mask)
            dst_v[slot, pl.ds(i * L, L)] = lax.shift_right_logical(v, sh)

    for b in range(NB):
        unpack(b, b)
        pltpu.async_copy(hs_hbm.at[src_v.at[b]], rows.at[b], gsem[b])

    def body(it, carry):
        j0 = it * NB
        for b in range(NB):
            j = j0 + b
            bp = (b + NB - 1) % NB
            pltpu.make_async_copy(
                hs_hbm.at[src_v.at[b]], rows.at[b], gsem[b]).wait()
            pltpu.async_copy(rows.at[b], acc.at[dst_v.at[b]], tsem[b],
                             add=True)

            @pl.when((j >= 1) & (j + NB - 1 < CT))
            def _():
                pltpu.make_async_copy(
                    rows.at[bp], acc.at[dst_v.at[bp]], tsem[bp]).wait()
                unpack(j + NB - 1, bp)
                pltpu.async_copy(
                    hs_hbm.at[src_v.at[bp]], rows.at[bp], gsem[bp])

        return carry

    lax.fori_loop(0, CT // NB, body, 0)
    for b in range(NB):
        pltpu.make_async_copy(
            rows.at[b], acc.at[dst_v.at[b]], tsem[b]).wait()
    plsc.subcore_barrier()
    pltpu.sync_copy(acc.at[pl.ds(s * RPT, RPT)],
                    out_hbm.at[c, pl.ds(s * RPT, RPT)])


BM = 256
GRID = NP // BM


def _tc_prep_body(x_ref, w_ref, dinv_ref, o_ref):
    h = jnp.dot(x_ref[...], w_ref[...], preferred_element_type=jnp.float32)
    o_ref[...] = h * dinv_ref[...]


def _tc_prep(x_pad, w, dinv_col):
    return pl.pallas_call(
        _tc_prep_body,
        grid=(GRID,),
        in_specs=[
            pl.BlockSpec((BM, D), lambda i: (i, 0)),
            pl.BlockSpec((D, D), lambda i: (0, 0)),
            pl.BlockSpec((BM, 1), lambda i: (i, 0)),
        ],
        out_specs=pl.BlockSpec((BM, D), lambda i: (i, 0)),
        out_shape=jax.ShapeDtypeStruct((NP, D), jnp.float32),
    )(x_pad, w, dinv_col)


def _tc_mid_body(pp_ref, hs_ref, dinv_ref, b_ref, w_ref, o_ref):
    t = pp_ref[0] + pp_ref[1] + hs_ref[...]
    t = jnp.maximum(t * dinv_ref[...] + b_ref[...], 0.0)
    h = jnp.dot(t, w_ref[...], preferred_element_type=jnp.float32)
    o_ref[...] = h * dinv_ref[...]


def _tc_mid(pp, hs, dinv_col, b_row, w):
    return pl.pallas_call(
        _tc_mid_body,
        grid=(GRID,),
        in_specs=[
            pl.BlockSpec((NC, BM, D), lambda i: (0, i, 0)),
            pl.BlockSpec((BM, D), lambda i: (i, 0)),
            pl.BlockSpec((BM, 1), lambda i: (i, 0)),
            pl.BlockSpec((1, D), lambda i: (0, 0)),
            pl.BlockSpec((D, D), lambda i: (0, 0)),
        ],
        out_specs=pl.BlockSpec((BM, D), lambda i: (i, 0)),
        out_shape=jax.ShapeDtypeStruct((NP, D), jnp.float32),
    )(pp, hs, dinv_col, b_row, w)


def _tc_final_body(pp_ref, hs_ref, dinv_ref, b_ref, o_ref):
    t = pp_ref[0] + pp_ref[1] + hs_ref[...]
    o_ref[...] = t * dinv_ref[...] + b_ref[...]


def _tc_final(pp, hs, dinv_col, b_row):
    return pl.pallas_call(
        _tc_final_body,
        grid=(GRID,),
        in_specs=[
            pl.BlockSpec((NC, BM, D), lambda i: (0, i, 0)),
            pl.BlockSpec((BM, D), lambda i: (i, 0)),
            pl.BlockSpec((BM, 1), lambda i: (i, 0)),
            pl.BlockSpec((1, D), lambda i: (0, 0)),
        ],
        out_specs=pl.BlockSpec((BM, D), lambda i: (i, 0)),
        out_shape=jax.ShapeDtypeStruct((NP, D), jnp.float32),
    )(pp, hs, dinv_col, b_row)


def kernel(x, edge_index, W1, b1, W2, b2):
    x_pad = jnp.pad(x, ((0, NP - N), (0, 0)))
    src = edge_index[0]
    dst = edge_index[1]
    dst3 = jnp.pad(dst, (0, EP - E), constant_values=N).reshape(NW, CT, B)
    packed = src | (dst << 14)
    pk3 = jnp.pad(packed, (0, EP - E),
                  constant_values=N << 14).reshape(NW, CT, B)
    zeros_blk = jnp.zeros((B, D), jnp.float32)

    degp = _sc_degree(dst3)
    deg = degp[0] + degp[1] + 1.0
    dinv_col = lax.rsqrt(deg).reshape(NP, 1)

    hs1 = _tc_prep(x_pad, W1, dinv_col)
    pp1 = _sc_prop(hs1, pk3, zeros_blk)
    hs2 = _tc_mid(pp1, hs1, dinv_col, b1.reshape(1, D), W2)
    pp2 = _sc_prop(hs2, pk3, zeros_blk)
    z = _tc_final(pp2, hs2, dinv_col, b2.reshape(1, D))
    return z[:N]


# speedup vs baseline: 1.0500x; 1.0500x over previous
import functools

import jax
import jax.numpy as jnp
from jax import lax
from jax.experimental import pallas as pl
from jax.experimental.pallas import tpu as pltpu
from jax.experimental.pallas import tpu_sc as plsc

N = 10000
E = 320000
D = 128

NC = 2
NS = 16
NW = NC * NS
B = 128
NB = 2
CT = NB * (-(-E // (NW * B * NB)))
EP = NW * B * CT
NP = 10240
RPT = NP // NS
L = 16

_MESH = plsc.VectorSubcoreMesh(core_axis_name="c", subcore_axis_name="s")


@functools.partial(
    pl.kernel,
    out_type=jax.ShapeDtypeStruct((NC, NP), jnp.float32),
    mesh=_MESH,
    scratch_types=[
        pltpu.VMEM((CT, B), jnp.int32),
        pltpu.VMEM((B,), jnp.float32),
        pltpu.VMEM((RPT,), jnp.float32),
        pltpu.VMEM_SHARED((NP,), jnp.float32),
    ],
)
def _sc_degree(dst_hbm, out_hbm, idx_v, ones_v, zrow_v, acc):
    c = lax.axis_index("c")
    s = lax.axis_index("s")
    wid = c * NS + s
    for i in range(B // L):
        ones_v[pl.ds(i * L, L)] = jnp.ones((L,), jnp.float32)
    for i in range(RPT // L):
        zrow_v[pl.ds(i * L, L)] = jnp.zeros((L,), jnp.float32)
    pltpu.sync_copy(zrow_v, acc.at[pl.ds(s * RPT, RPT)])
    pltpu.sync_copy(dst_hbm.at[wid], idx_v)
    plsc.subcore_barrier()

    def body(j, carry):
        pltpu.sync_copy(ones_v, acc.at[idx_v.at[j]], add=True)
        return carry

    lax.fori_loop(0, CT, body, 0)
    plsc.subcore_barrier()
    pltpu.sync_copy(acc.at[pl.ds(s * RPT, RPT)],
                    out_hbm.at[c, pl.ds(s * RPT, RPT)])


@functools.partial(
    pl.kernel,
    out_type=jax.ShapeDtypeStruct((NC, NP, D), jnp.float32),
    mesh=_MESH,
    scratch_types=[
        pltpu.VMEM((CT, B), jnp.int32),
        pltpu.VMEM((NB, B), jnp.int32),
        pltpu.VMEM((NB, B), jnp.int32),
        pltpu.VMEM((NB, B, D), jnp.float32),
        pltpu.VMEM_SHARED((NP, D), jnp.float32),
        pltpu.SemaphoreType.DMA,
        pltpu.SemaphoreType.DMA,
        pltpu.SemaphoreType.DMA,
        pltpu.SemaphoreType.DMA,
    ],
)
def _sc_prop(hs_hbm, pk_hbm, zeros_hbm, out_hbm,
             pk_v, src_v, dst_v, rows, acc,
             g0, g1, t0, t1):
    c = lax.axis_index("c")
    s = lax.axis_index("s")
    wid = c * NS + s
    gsem = (g0, g1)
    tsem = (t0, t1)

    pltpu.sync_copy(zeros_hbm, rows.at[0])
    for r in range(RPT // B):
        pltpu.sync_copy(rows.at[0], acc.at[pl.ds(s * RPT + r * B, B)])
    pltpu.sync_copy(pk_hbm.at[wid], pk_v)
    plsc.subcore_barrier()

    def unpack(j, slot):
        mask = jnp.full((L,), 0x3FFF, jnp.int32)
        sh = jnp.full((L,), 14, jnp.int32)
        for i in range(B // L):
            v = pk_v[j, pl.ds(i * L, L)]
            src_v[slot, pl.ds(i * L, L)] = lax.bitwise_and(v, mask)
            dst_v[slot, pl.ds(i * L, L)] = lax.shift_right_logical(v, sh)

    for b in range(NB):
        unpack(b, b)
        pltpu.async_copy(hs_hbm.at[src_v.at[b]], rows.at[b], gsem[b])

    def body(it, carry):
        j0 = it * NB
        for b in range(NB):
            j = j0 + b
            bn = (b + 1) % NB

            @pl.when((j >= 1) & (j + 1 < CT))
            def _():
                pltpu.make_async_copy(
                    rows.at[bn], acc.at[dst_v.at[bn]], tsem[bn]).wait()
                unpack(j + 1, bn)
                pltpu.async_copy(
                    hs_hbm.at[src_v.at[bn]], rows.at[bn], gsem[bn])

            pltpu.make_async_copy(
                hs_hbm.at[src_v.at[b]], rows.at[b], gsem[b]).wait()
            pltpu.async_copy(rows.at[b], acc.at[dst_v.at[b]], tsem[b],
                             add=True)

        return carry

    lax.fori_loop(0, CT // NB, body, 0)
    for b in range(NB):
        pltpu.make_async_copy(
            rows.at[b], acc.at[dst_v.at[b]], tsem[b]).wait()
    plsc.subcore_barrier()
    pltpu.sync_copy(acc.at[pl.ds(s * RPT, RPT)],
                    out_hbm.at[c, pl.ds(s * RPT, RPT)])


BM = 256
GRID = NP // BM


def _tc_prep_body(x_ref, w_ref, dinv_ref, o_ref):
    h = jnp.dot(x_ref[...], w_ref[...], preferred_element_type=jnp.float32)
    o_ref[...] = h * dinv_ref[...]


def _tc_prep(x_pad, w, dinv_col):
    return pl.pallas_call(
        _tc_prep_body,
        grid=(GRID,),
        in_specs=[
            pl.BlockSpec((BM, D), lambda i: (i, 0)),
            pl.BlockSpec((D, D), lambda i: (0, 0)),
            pl.BlockSpec((BM, 1), lambda i: (i, 0)),
        ],
        out_specs=pl.BlockSpec((BM, D), lambda i: (i, 0)),
        out_shape=jax.ShapeDtypeStruct((NP, D), jnp.float32),
    )(x_pad, w, dinv_col)


def _tc_mid_body(pp_ref, hs_ref, dinv_ref, b_ref, w_ref, o_ref):
    t = pp_ref[0] + pp_ref[1] + hs_ref[...]
    t = jnp.maximum(t * dinv_ref[...] + b_ref[...], 0.0)
    h = jnp.dot(t, w_ref[...], preferred_element_type=jnp.float32)
    o_ref[...] = h * dinv_ref[...]


def _tc_mid(pp, hs, dinv_col, b_row, w):
    return pl.pallas_call(
        _tc_mid_body,
        grid=(GRID,),
        in_specs=[
            pl.BlockSpec((NC, BM, D), lambda i: (0, i, 0)),
            pl.BlockSpec((BM, D), lambda i: (i, 0)),
            pl.BlockSpec((BM, 1), lambda i: (i, 0)),
            pl.BlockSpec((1, D), lambda i: (0, 0)),
            pl.BlockSpec((D, D), lambda i: (0, 0)),
        ],
        out_specs=pl.BlockSpec((BM, D), lambda i: (i, 0)),
        out_shape=jax.ShapeDtypeStruct((NP, D), jnp.float32),
    )(pp, hs, dinv_col, b_row, w)


def _tc_final_body(pp_ref, hs_ref, dinv_ref, b_ref, o_ref):
    t = pp_ref[0] + pp_ref[1] + hs_ref[...]
    o_ref[...] = t * dinv_ref[...] + b_ref[...]


def _tc_final(pp, hs, dinv_col, b_row):
    return pl.pallas_call(
        _tc_final_body,
        grid=(GRID,),
        in_specs=[
            pl.BlockSpec((NC, BM, D), lambda i: (0, i, 0)),
            pl.BlockSpec((BM, D), lambda i: (i, 0)),
            pl.BlockSpec((BM, 1), lambda i: (i, 0)),
            pl.BlockSpec((1, D), lambda i: (0, 0)),
        ],
        out_specs=pl.BlockSpec((BM, D), lambda i: (i, 0)),
        out_shape=jax.ShapeDtypeStruct((NP, D), jnp.float32),
    )(pp, hs, dinv_col, b_row)


def kernel(x, edge_index, W1, b1, W2, b2):
    x_pad = jnp.pad(x, ((0, NP - N), (0, 0)))
    src = edge_index[0]
    dst = edge_index[1]
    dst3 = jnp.pad(dst, (0, EP - E), constant_values=N).reshape(NW, CT, B)
    packed = src | (dst << 14)
    pk3 = jnp.pad(packed, (0, EP - E),
                  constant_values=N << 14).reshape(NW, CT, B)
    zeros_blk = jnp.zeros((B, D), jnp.float32)

    degp = _sc_degree(dst3)
    deg = degp[0] + degp[1] + 1.0
    dinv_col = lax.rsqrt(deg).reshape(NP, 1)

    hs1 = _tc_prep(x_pad, W1, dinv_col)
    pp1 = _sc_prop(hs1, pk3, zeros_blk)
    hs2 = _tc_mid(pp1, hs1, dinv_col, b1.reshape(1, D), W2)
    pp2 = _sc_prop(hs2, pk3, zeros_blk)
    z = _tc_final(pp2, hs2, dinv_col, b2.reshape(1, D))
    return z[:N]


# speedup vs baseline: 2.8975x; 2.7597x over previous
import functools

import jax
import jax.numpy as jnp
from jax import lax
from jax.experimental import pallas as pl
from jax.experimental.pallas import tpu as pltpu
from jax.experimental.pallas import tpu_sc as plsc

N = 10000
E = 320000
D = 128

NC = 2
NS = 16
NW = NC * NS
B = 128
CT = 2 * (-(-E // (NW * B * 2)))
EP = NW * B * CT
NP = 10240
RPT = NP // NS
L = 16

_MESH = plsc.VectorSubcoreMesh(core_axis_name="c", subcore_axis_name="s")


@functools.partial(
    pl.kernel,
    out_type=jax.ShapeDtypeStruct((NC, NP), jnp.float32),
    mesh=_MESH,
    scratch_types=[
        pltpu.VMEM((CT, B), jnp.int32),
        pltpu.VMEM((B,), jnp.float32),
        pltpu.VMEM((RPT,), jnp.float32),
        pltpu.VMEM_SHARED((NP,), jnp.float32),
    ],
)
def _sc_degree(dst_hbm, out_hbm, idx_v, ones_v, zrow_v, acc):
    c = lax.axis_index("c")
    s = lax.axis_index("s")
    wid = c * NS + s
    for i in range(B // L):
        ones_v[pl.ds(i * L, L)] = jnp.ones((L,), jnp.float32)
    for i in range(RPT // L):
        zrow_v[pl.ds(i * L, L)] = jnp.zeros((L,), jnp.float32)
    pltpu.sync_copy(zrow_v, acc.at[pl.ds(s * RPT, RPT)])
    pltpu.sync_copy(dst_hbm.at[wid], idx_v)
    plsc.subcore_barrier()

    def body(j, carry):
        pltpu.sync_copy(ones_v, acc.at[idx_v.at[j]], add=True)
        return carry

    lax.fori_loop(0, CT, body, 0)
    plsc.subcore_barrier()
    pltpu.sync_copy(acc.at[pl.ds(s * RPT, RPT)],
                    out_hbm.at[c, pl.ds(s * RPT, RPT)])


@functools.partial(
    pl.kernel,
    out_type=jax.ShapeDtypeStruct((NC, NP, D), jnp.float32),
    mesh=_MESH,
    scratch_types=[
        pltpu.VMEM((CT, B), jnp.int32),
        pltpu.VMEM((2, B), jnp.int32),
        pltpu.VMEM((2, B), jnp.int32),
        pltpu.VMEM((B, D), jnp.float32),
        pltpu.VMEM((B, D), jnp.float32),
        pltpu.VMEM_SHARED((NP, D), jnp.float32),
        pltpu.SemaphoreType.DMA,
        pltpu.SemaphoreType.DMA,
    ],
)
def _sc_prop(hs_hbm, pk_hbm, zeros_hbm, out_hbm,
             pk_v, src_v, dst_v, rows_a, rows_b, acc, sem_a, sem_b):
    c = lax.axis_index("c")
    s = lax.axis_index("s")
    wid = c * NS + s
    pltpu.sync_copy(zeros_hbm, acc.at[pl.ds(s * RPT, RPT)])
    pltpu.sync_copy(pk_hbm.at[wid], pk_v)
    plsc.subcore_barrier()

    def unpack(j, slot):
        mask = jnp.full((L,), 0x3FFF, jnp.int32)
        sh = jnp.full((L,), 14, jnp.int32)
        for i in range(B // L):
            v = pk_v[j, pl.ds(i * L, L)]
            src_v[slot, pl.ds(i * L, L)] = lax.bitwise_and(v, mask)
            dst_v[slot, pl.ds(i * L, L)] = lax.shift_right_logical(v, sh)

    unpack(0, 0)
    pltpu.async_copy(hs_hbm.at[src_v.at[0]], rows_a, sem_a)

    def body(it, carry):
        j = it * 2
        unpack(j + 1, 1)
        pltpu.async_copy(hs_hbm.at[src_v.at[1]], rows_b, sem_b)
        pltpu.make_async_copy(hs_hbm.at[src_v.at[0]], rows_a, sem_a).wait()
        pltpu.sync_copy(rows_a, acc.at[dst_v.at[0]], add=True)

        @pl.when(j + 2 < CT)
        def _():
            unpack(j + 2, 0)
            pltpu.async_copy(hs_hbm.at[src_v.at[0]], rows_a, sem_a)

        pltpu.make_async_copy(hs_hbm.at[src_v.at[1]], rows_b, sem_b).wait()
        pltpu.sync_copy(rows_b, acc.at[dst_v.at[1]], add=True)
        return carry

    lax.fori_loop(0, CT // 2, body, 0)
    plsc.subcore_barrier()
    pltpu.sync_copy(acc.at[pl.ds(s * RPT, RPT)],
                    out_hbm.at[c, pl.ds(s * RPT, RPT)])


BM = 256
GRID = NP // BM


def _tc_prep_body(x_ref, w_ref, dinv_ref, o_ref):
    h = jnp.dot(x_ref[...], w_ref[...], preferred_element_type=jnp.float32)
    o_ref[...] = h * dinv_ref[...]


def _tc_prep(x_pad, w, dinv_col):
    return pl.pallas_call(
        _tc_prep_body,
        grid=(GRID,),
        in_specs=[
            pl.BlockSpec((BM, D), lambda i: (i, 0)),
            pl.BlockSpec((D, D), lambda i: (0, 0)),
            pl.BlockSpec((BM, 1), lambda i: (i, 0)),
        ],
        out_specs=pl.BlockSpec((BM, D), lambda i: (i, 0)),
        out_shape=jax.ShapeDtypeStruct((NP, D), jnp.float32),
    )(x_pad, w, dinv_col)


def _tc_mid_body(pp_ref, hs_ref, dinv_ref, b_ref, w_ref, o_ref):
    t = pp_ref[0] + pp_ref[1] + hs_ref[...]
    t = jnp.maximum(t * dinv_ref[...] + b_ref[...], 0.0)
    h = jnp.dot(t, w_ref[...], preferred_element_type=jnp.float32)
    o_ref[...] = h * dinv_ref[...]


def _tc_mid(pp, hs, dinv_col, b_row, w):
    return pl.pallas_call(
        _tc_mid_body,
        grid=(GRID,),
        in_specs=[
            pl.BlockSpec((NC, BM, D), lambda i: (0, i, 0)),
            pl.BlockSpec((BM, D), lambda i: (i, 0)),
            pl.BlockSpec((BM, 1), lambda i: (i, 0)),
            pl.BlockSpec((1, D), lambda i: (0, 0)),
            pl.BlockSpec((D, D), lambda i: (0, 0)),
        ],
        out_specs=pl.BlockSpec((BM, D), lambda i: (i, 0)),
        out_shape=jax.ShapeDtypeStruct((NP, D), jnp.float32),
    )(pp, hs, dinv_col, b_row, w)


def _tc_final_body(pp_ref, hs_ref, dinv_ref, b_ref, o_ref):
    t = pp_ref[0] + pp_ref[1] + hs_ref[...]
    o_ref[...] = t * dinv_ref[...] + b_ref[...]


def _tc_final(pp, hs, dinv_col, b_row):
    return pl.pallas_call(
        _tc_final_body,
        grid=(GRID,),
        in_specs=[
            pl.BlockSpec((NC, BM, D), lambda i: (0, i, 0)),
            pl.BlockSpec((BM, D), lambda i: (i, 0)),
            pl.BlockSpec((BM, 1), lambda i: (i, 0)),
            pl.BlockSpec((1, D), lambda i: (0, 0)),
        ],
        out_specs=pl.BlockSpec((BM, D), lambda i: (i, 0)),
        out_shape=jax.ShapeDtypeStruct((NP, D), jnp.float32),
    )(pp, hs, dinv_col, b_row)


def kernel(x, edge_index, W1, b1, W2, b2):
    x_pad = jnp.pad(x, ((0, NP - N), (0, 0)))
    src = edge_index[0]
    dst = edge_index[1]
    pad_i = jnp.arange(EP - E, dtype=jnp.int32)
    pad_dst = N + pad_i % (NP - N)
    pad_src = pad_i % N
    dst3 = jnp.concatenate([dst, pad_dst]).reshape(NW, CT, B)
    pk3 = jnp.concatenate(
        [src | (dst << 14), pad_src | (pad_dst << 14)]).reshape(NW, CT, B)
    zeros_rows = jnp.zeros((RPT, D), jnp.float32)

    degp = _sc_degree(dst3)
    deg = degp[0] + degp[1] + 1.0
    dinv_col = lax.rsqrt(deg).reshape(NP, 1)

    hs1 = _tc_prep(x_pad, W1, dinv_col)
    pp1 = _sc_prop(hs1, pk3, zeros_rows)
    hs2 = _tc_mid(pp1, hs1, dinv_col, b1.reshape(1, D), W2)
    pp2 = _sc_prop(hs2, pk3, zeros_rows)
    z = _tc_final(pp2, hs2, dinv_col, b2.reshape(1, D))
    return z[:N]
